# baseline (speedup 1.0000x reference)
import functools

import jax
import jax.numpy as jnp
from jax import lax
from jax.experimental import pallas as pl
from jax.experimental.pallas import tpu as pltpu
from jax.experimental.pallas import tpu_sc as plsc

B, Q, C = 8, 1000, 4096
QPW = 256
QSTEP = 248
NG = QPW // 16
NEG = -3.402823e38
FB, FL, FC = 0, QPW, 2 * QPW


def _body(bb_h, lb_h, ch_h, blg_h, llg_h, cl_h, ts_h, out_h,
          comb, blgv, llgv, tsv, outv, buf2, rowbuf, flags, sem):
    wid = lax.axis_index("s") * 2 + lax.axis_index("c")
    img = wid // 4
    qs = (wid % 4) * QSTEP
    q_sl = pl.ds(qs, QPW)

    cps = [
        pltpu.async_copy(cl_h.at[img, pl.ds(qs // 8, QPW // 8), 0], buf2, sem),
        pltpu.async_copy(bb_h.at[img, :, q_sl], comb.at[pl.ds(0, 4)], sem),
        pltpu.async_copy(lb_h.at[img, :, q_sl], comb.at[pl.ds(4, 4)], sem),
        pltpu.async_copy(ch_h.at[img, :, q_sl], comb.at[pl.ds(8, 16)], sem),
        pltpu.async_copy(blg_h.at[img, :, q_sl], blgv, sem),
        pltpu.async_copy(llg_h.at[img, :, q_sl], llgv, sem),
        pltpu.async_copy(ts_h, tsv.at[:, pl.ds(0, 8)], sem),
    ]
    for cp in cps:
        cp.wait()

    lanes = lax.iota(jnp.int32, 16)

    def char_group(gi, _):
        rows = gi * 16 + lanes
        tq = rows // 8
        qi = rows % 8
        v0 = plsc.load_gather(buf2, [tq, qi, lanes * 0])
        acc = v0 != v0
        for c in range(1, 128):
            acc = jnp.logical_or(
                acc, plsc.load_gather(buf2, [tq, qi, lanes * 0 + c]) > v0)
        flags[pl.ds(FC + gi * 16, 16)] = jnp.where(acc, 1.0, -1.0)
        return 0

    lax.fori_loop(0, NG, char_group, 0)

    def bl_group(gi, _):
        g_sl = pl.ds(gi * 16, 16)
        vb0 = blgv[0, g_sl]
        vl0 = llgv[0, g_sl]
        ab = vb0 != vb0
        al = ab
        for c in range(1, 16):
            ab = jnp.logical_or(ab, blgv[c, g_sl] > vb0)
            al = jnp.logical_or(al, llgv[c, g_sl] > vl0)
        flags[pl.ds(FB + gi * 16, 16)] = jnp.where(ab, 1.0, 0.0)
        flags[pl.ds(FL + gi * 16, 16)] = jnp.where(al, 1.0, 0.0)
        return 0

    lax.fori_loop(0, NG, bl_group, 0)

    def resolve_group(gi, _):
        fvec = flags[pl.ds(FC + gi * 16, 16)]

        @pl.when(jnp.min(fvec) < 0.0)
        def _():
            def resolve(rr, _):
                r = gi * 16 + rr
                fc_here = plsc.load_gather(flags, [FC + r + lanes * 0])

                @pl.when(fc_here[0] < 0.0)
                def _():
                    qi = r % 8
                    pltpu.sync_copy(cl_h.at[img, qs // 8 + r // 8], rowbuf)
                    accs = tuple(rowbuf[1, qi, pl.ds(k * 16, 16)]
                                 for k in range(8))

                    def chunk(t, a):
                        return tuple(
                            jnp.maximum(a[k], rowbuf[t, qi, pl.ds(k * 16, 16)])
                            for k in range(8))

                    accs = lax.fori_loop(2, 32, chunk, accs)
                    m = accs[0]
                    for k in range(1, 8):
                        m = jnp.maximum(m, accs[k])
                    v0v = plsc.load_gather(
                        buf2, [lanes * 0 + r // 8, lanes * 0 + r % 8, lanes * 0])
                    val = jnp.where(jnp.max(m) > v0v[0], 1.0, 0.0) + lanes * 0.0
                    plsc.store_scatter(flags, [FC + r + lanes * 0], val,
                                       mask=lanes == 0)
                return 0

            lax.fori_loop(0, 16, resolve, 0)
        return 0

    lax.fori_loop(0, NG, resolve_group, 0)

    hsp = plsc.load_gather(tsv, [lanes * 0, lanes * 0 + img])
    wsp = plsc.load_gather(tsv, [lanes * 0 + 1, lanes * 0 + img])

    def emit(gi, _):
        g_sl = pl.ds(gi * 16, 16)
        fb = flags[pl.ds(FB + gi * 16, 16)]
        fl = flags[pl.ds(FL + gi * 16, 16)]
        fc = flags[pl.ds(FC + gi * 16, 16)]
        for base, f in ((0, fb), (4, fl)):
            cx = comb[base + 0, g_sl]
            cy = comb[base + 1, g_sl]
            hw = comb[base + 2, g_sl] * 0.5
            hh = comb[base + 3, g_sl] * 0.5
            outv[base + 0, g_sl] = (cx - hw) * wsp * f
            outv[base + 1, g_sl] = (cy - hh) * hsp * f
            outv[base + 2, g_sl] = (cx + hw) * wsp * f
            outv[base + 3, g_sl] = (cy + hh) * hsp * f
        for c in range(16):
            sc = hsp if c % 2 == 0 else wsp
            outv[8 + c, g_sl] = comb[8 + c, g_sl] * sc * fc
        return 0

    lax.fori_loop(0, NG, emit, 0)

    pltpu.sync_copy(outv, out_h.at[img, :, q_sl])


@jax.jit
def kernel(pred_block, pred_line, pred_char, pred_block_logits,
           pred_line_logits, pred_char_logits, target_sizes):
    mesh = plsc.VectorSubcoreMesh(core_axis_name="c", subcore_axis_name="s")
    run = functools.partial(
        pl.kernel,
        mesh=mesh,
        compiler_params=pltpu.CompilerParams(
            needs_layout_passes=False, use_tc_tiling_on_sc=False),
        out_type=jax.ShapeDtypeStruct((B, 24, Q), jnp.float32),
        scratch_types=[
            pltpu.VMEM((24, QPW), jnp.float32),
            pltpu.VMEM((16, QPW), jnp.float32),
            pltpu.VMEM((16, QPW), jnp.float32),
            pltpu.VMEM((2, 128), jnp.float32),
            pltpu.VMEM((24, QPW), jnp.float32),
            pltpu.VMEM((QPW // 8, 8, 128), jnp.float32),
            pltpu.VMEM((32, 8, 128), jnp.float32),
            pltpu.VMEM((3 * QPW,), jnp.float32),
            pltpu.SemaphoreType.DMA,
        ],
    )(_body)
    out = run(pred_block.transpose(0, 2, 1), pred_line.transpose(0, 2, 1),
              pred_char.transpose(0, 2, 1),
              pred_block_logits.transpose(0, 2, 1),
              pred_line_logits.transpose(0, 2, 1),
              pred_char_logits.reshape(B, Q // 8, 8, C // 128, 128)
                              .transpose(0, 1, 3, 2, 4),
              target_sizes.transpose(1, 0))
    return out.transpose(0, 2, 1)


# speedup vs baseline: 1.2532x; 1.2532x over previous
import functools

import jax
import jax.numpy as jnp
from jax import lax
from jax.experimental import pallas as pl
from jax.experimental.pallas import tpu as pltpu
from jax.experimental.pallas import tpu_sc as plsc

B, Q, C = 8, 1000, 4096
QPW = 256
QSTEP = 248
NG = QPW // 16


def _sc_body(cl_h, out_h, buf2, rowbuf, flags, sem):
    wid = lax.axis_index("s") * 2 + lax.axis_index("c")
    img = wid // 4
    qs = (wid % 4) * QSTEP

    pltpu.async_copy(
        cl_h.at[img, pl.ds(qs // 8, QPW // 8), 0], buf2, sem).wait()

    lanes = lax.iota(jnp.int32, 16)

    def char_group(gi, _):
        rows = gi * 16 + lanes
        tq = rows // 8
        qi = rows % 8
        v0 = plsc.load_gather(buf2, [tq, qi, lanes * 0])
        acc = v0 != v0
        for c in range(1, 128):
            acc = jnp.logical_or(
                acc, plsc.load_gather(buf2, [tq, qi, lanes * 0 + c]) > v0)
        flags[pl.ds(gi * 16, 16)] = jnp.where(acc, 1.0, -1.0)
        return 0

    lax.fori_loop(0, NG, char_group, 0)

    def resolve_group(gi, _):
        fvec = flags[pl.ds(gi * 16, 16)]

        @pl.when(jnp.min(fvec) < 0.0)
        def _():
            def resolve(rr, _):
                r = gi * 16 + rr
                fc_here = plsc.load_gather(flags, [r + lanes * 0])

                @pl.when(fc_here[0] < 0.0)
                def _():
                    pltpu.sync_copy(
                        cl_h.at[img, qs // 8 + r // 8, pl.ds(1, 31), r % 8],
                        rowbuf)
                    accs = tuple(rowbuf[0, pl.ds(k * 16, 16)] for k in range(8))

                    def chunk(t, a):
                        return tuple(
                            jnp.maximum(a[k], rowbuf[t, pl.ds(k * 16, 16)])
                            for k in range(8))

                    accs = lax.fori_loop(1, 31, chunk, accs)
                    m = accs[0]
                    for k in range(1, 8):
                        m = jnp.maximum(m, accs[k])
                    v0v = plsc.load_gather(
                        buf2, [lanes * 0 + r // 8, lanes * 0 + r % 8, lanes * 0])
                    val = jnp.where(jnp.max(m) > v0v[0], 1.0, 0.0) + lanes * 0.0
                    plsc.store_scatter(flags, [r + lanes * 0], val,
                                       mask=lanes == 0)
                return 0

            lax.fori_loop(0, 16, resolve, 0)
        return 0

    lax.fori_loop(0, NG, resolve_group, 0)

    pltpu.sync_copy(flags, out_h.at[pl.ds(img * 1024 + qs, QPW)])


def _tc_body(ts, bb, lb, ch, blg, llg, cf, out):
    b = pl.program_id(0)
    tcols = lax.broadcasted_iota(jnp.int32, (2, 8), 1)
    tsb = jnp.sum(jnp.where(tcols == b, ts[...], 0.0), axis=1)
    h = tsb[0]
    w = tsb[1]

    def keep(ref):
        v0 = ref[0, 0, :]
        acc = ref[0, 1, :] > v0
        for c in range(2, 16):
            acc = jnp.logical_or(acc, ref[0, c, :] > v0)
        return jnp.where(acc, 1.0, 0.0)

    fb = keep(blg)
    fl = keep(llg)
    crows = lax.broadcasted_iota(jnp.int32, (8, 1024), 0)
    fc = jnp.sum(jnp.where(crows == b, cf[...], 0.0), axis=0)[:Q]
    for base, src, f in ((0, bb, fb), (4, lb, fl)):
        cx = src[0, 0, :]
        cy = src[0, 1, :]
        hw = src[0, 2, :] * 0.5
        hh = src[0, 3, :] * 0.5
        out[0, base + 0, :] = (cx - hw) * w * f
        out[0, base + 1, :] = (cy - hh) * h * f
        out[0, base + 2, :] = (cx + hw) * w * f
        out[0, base + 3, :] = (cy + hh) * h * f
    for c in range(16):
        s = h if c % 2 == 0 else w
        out[0, 8 + c, :] = ch[0, c, :] * s * fc


@jax.jit
def kernel(pred_block, pred_line, pred_char, pred_block_logits,
           pred_line_logits, pred_char_logits, target_sizes):
    mesh = plsc.VectorSubcoreMesh(core_axis_name="c", subcore_axis_name="s")
    sc_run = functools.partial(
        pl.kernel,
        mesh=mesh,
        compiler_params=pltpu.CompilerParams(
            needs_layout_passes=False, use_tc_tiling_on_sc=False),
        out_type=jax.ShapeDtypeStruct((B * 1024,), jnp.float32),
        scratch_types=[
            pltpu.VMEM((QPW // 8, 8, 128), jnp.float32),
            pltpu.VMEM((31, 128), jnp.float32),
            pltpu.VMEM((QPW,), jnp.float32),
            pltpu.SemaphoreType.DMA,
        ],
    )(_sc_body)
    cflags = sc_run(
        pred_char_logits.reshape(B, Q // 8, 8, C // 128, 128)
                        .transpose(0, 1, 3, 2, 4)).reshape(B, 1024)

    v3 = lambda: pl.BlockSpec((1, None, Q), lambda b: (b, 0, 0))
    out = pl.pallas_call(
        _tc_body,
        grid=(B,),
        in_specs=[
            pl.BlockSpec((2, 8), lambda b: (0, 0)),
            pl.BlockSpec((1, 4, Q), lambda b: (b, 0, 0)),
            pl.BlockSpec((1, 4, Q), lambda b: (b, 0, 0)),
            pl.BlockSpec((1, 16, Q), lambda b: (b, 0, 0)),
            pl.BlockSpec((1, 16, Q), lambda b: (b, 0, 0)),
            pl.BlockSpec((1, 16, Q), lambda b: (b, 0, 0)),
            pl.BlockSpec((8, 1024), lambda b: (0, 0)),
        ],
        out_specs=pl.BlockSpec((1, 24, Q), lambda b: (b, 0, 0)),
        out_shape=jax.ShapeDtypeStruct((B, 24, Q), jnp.float32),
    )(target_sizes.transpose(1, 0),
      pred_block.transpose(0, 2, 1), pred_line.transpose(0, 2, 1),
      pred_char.transpose(0, 2, 1),
      pred_block_logits.transpose(0, 2, 1),
      pred_line_logits.transpose(0, 2, 1),
      cflags)
    return out.transpose(0, 2, 1)


# speedup vs baseline: 1.3818x; 1.1026x over previous
import functools

import jax
import jax.numpy as jnp
from jax import lax
from jax.experimental import pallas as pl
from jax.experimental.pallas import tpu as pltpu
from jax.experimental.pallas import tpu_sc as plsc

B, Q, C = 8, 1000, 4096
QPW = 256
QSTEP = 248
NG = QPW // 16


def _sc_body(cl_h, out_h, buf2, rowbuf, flags, sem):
    wid = lax.axis_index("s") * 2 + lax.axis_index("c")
    img = wid // 4
    qs = (wid % 4) * QSTEP

    pltpu.async_copy(
        cl_h.at[img, pl.ds(qs // 8, QPW // 8), 0], buf2, sem).wait()

    lanes = lax.iota(jnp.int32, 16)

    zero16 = lanes * 0

    def char_group(gi, _):
        rows = gi * 16 + lanes
        tq = rows // 8
        qi = rows % 8
        v0 = plsc.load_gather(buf2, [tq, qi, zero16])
        accs = [v0 != v0] * 4
        for c0 in range(0, 128, 4):
            for k in range(4):
                cvec = jnp.full((16,), c0 + k, jnp.int32)
                accs[k] = jnp.logical_or(
                    accs[k], plsc.load_gather(buf2, [tq, qi, cvec]) > v0)
        acc = jnp.logical_or(jnp.logical_or(accs[0], accs[1]),
                             jnp.logical_or(accs[2], accs[3]))
        flags[pl.ds(gi * 16, 16)] = jnp.where(acc, 1.0, -1.0)
        return 0

    lax.fori_loop(0, NG, char_group, 0)

    def resolve_group(gi, _):
        fvec = flags[pl.ds(gi * 16, 16)]

        @pl.when(jnp.min(fvec) < 0.0)
        def _():
            def resolve(rr, _):
                r = gi * 16 + rr
                fc_here = plsc.load_gather(flags, [r + lanes * 0])

                @pl.when(fc_here[0] < 0.0)
                def _():
                    pltpu.sync_copy(
                        cl_h.at[img, qs // 8 + r // 8, pl.ds(1, 31), r % 8],
                        rowbuf)
                    accs = tuple(rowbuf[0, pl.ds(k * 16, 16)] for k in range(8))

                    def chunk(t, a):
                        return tuple(
                            jnp.maximum(a[k], rowbuf[t, pl.ds(k * 16, 16)])
                            for k in range(8))

                    accs = lax.fori_loop(1, 31, chunk, accs)
                    m = accs[0]
                    for k in range(1, 8):
                        m = jnp.maximum(m, accs[k])
                    v0v = plsc.load_gather(
                        buf2, [lanes * 0 + r // 8, lanes * 0 + r % 8, lanes * 0])
                    val = jnp.where(jnp.max(m) > v0v[0], 1.0, 0.0) + lanes * 0.0
                    plsc.store_scatter(flags, [r + lanes * 0], val,
                                       mask=lanes == 0)
                return 0

            lax.fori_loop(0, 16, resolve, 0)
        return 0

    lax.fori_loop(0, NG, resolve_group, 0)

    pltpu.sync_copy(flags, out_h.at[pl.ds(img * 1024 + qs, QPW)])


def _tc_body(ts, bb, lb, ch, blg, llg, cf, out):
    for b in range(B):
        h = ts[0, b]
        w = ts[1, b]

        def keep(ref, b=b):
            v0 = ref[b, 0, :]
            acc = ref[b, 1, :] > v0
            for c in range(2, 16):
                acc = jnp.logical_or(acc, ref[b, c, :] > v0)
            return jnp.where(acc, 1.0, 0.0)

        fb = keep(blg)
        fl = keep(llg)
        fc = cf[8 * b:8 * b + 8, :].reshape(1024)[:Q]
        for base, src, f in ((0, bb, fb), (4, lb, fl)):
            cx = src[b, 0, :]
            cy = src[b, 1, :]
            hw = src[b, 2, :] * 0.5
            hh = src[b, 3, :] * 0.5
            out[b, base + 0, :] = (cx - hw) * w * f
            out[b, base + 1, :] = (cy - hh) * h * f
            out[b, base + 2, :] = (cx + hw) * w * f
            out[b, base + 3, :] = (cy + hh) * h * f
        for c in range(16):
            s = h if c % 2 == 0 else w
            out[b, 8 + c, :] = ch[b, c, :] * s * fc


@jax.jit
def kernel(pred_block, pred_line, pred_char, pred_block_logits,
           pred_line_logits, pred_char_logits, target_sizes):
    mesh = plsc.VectorSubcoreMesh(core_axis_name="c", subcore_axis_name="s")
    sc_run = functools.partial(
        pl.kernel,
        mesh=mesh,
        compiler_params=pltpu.CompilerParams(
            needs_layout_passes=False, use_tc_tiling_on_sc=False),
        out_type=jax.ShapeDtypeStruct((B * 1024,), jnp.float32),
        scratch_types=[
            pltpu.VMEM((QPW // 8, 8, 128), jnp.float32),
            pltpu.VMEM((31, 128), jnp.float32),
            pltpu.VMEM((QPW,), jnp.float32),
            pltpu.SemaphoreType.DMA,
        ],
    )(_sc_body)
    cflags = sc_run(
        pred_char_logits.reshape(B, Q // 8, 8, C // 128, 128)
                        .transpose(0, 1, 3, 2, 4)).reshape(B * 8, 128)

    out = pl.pallas_call(
        _tc_body,
        out_shape=jax.ShapeDtypeStruct((B, 24, Q), jnp.float32),
    )(target_sizes.transpose(1, 0),
      pred_block.transpose(0, 2, 1), pred_line.transpose(0, 2, 1),
      pred_char.transpose(0, 2, 1),
      pred_block_logits.transpose(0, 2, 1),
      pred_line_logits.transpose(0, 2, 1),
      cflags)
    return out.transpose(0, 2, 1)


# speedup vs baseline: 1.8059x; 1.3069x over previous
import functools

import jax
import jax.numpy as jnp
from jax import lax
from jax.experimental import pallas as pl
from jax.experimental.pallas import tpu as pltpu
from jax.experimental.pallas import tpu_sc as plsc

B, Q, C = 8, 1000, 4096
QPW = 256
QSTEP = 248
NG = QPW // 16


def _sc_body(cl_h, out_h, buf2, rowbuf, flags, sem):
    wid = lax.axis_index("s") * 2 + lax.axis_index("c")
    img = wid // 4
    qs = (wid % 4) * QSTEP

    pltpu.async_copy(
        cl_h.at[img, pl.ds(qs // 8, QPW // 8), 0],
        buf2.at[:, :, pl.ds(0, 128)], sem).wait()

    lanes = lax.iota(jnp.int32, 16)

    zero16 = lanes * 0

    def char_group(gi, _):
        rows = gi * 16 + lanes
        tq = rows // 8
        qi = rows % 8
        v0 = plsc.load_gather(buf2, [tq, qi, zero16])
        accs = [v0 != v0] * 4
        for c0 in range(0, 128, 4):
            for k in range(4):
                cvec = jnp.full((16,), c0 + k, jnp.int32)
                accs[k] = jnp.logical_or(
                    accs[k], plsc.load_gather(buf2, [tq, qi, cvec]) > v0)
        acc = jnp.logical_or(jnp.logical_or(accs[0], accs[1]),
                             jnp.logical_or(accs[2], accs[3]))
        flags[pl.ds(gi * 16, 16)] = jnp.where(acc, 1.0, -1.0)
        return 0

    lax.fori_loop(0, NG, char_group, 0)

    def resolve_group(gi, _):
        fvec = flags[pl.ds(gi * 16, 16)]

        @pl.when(jnp.min(fvec) < 0.0)
        def _():
            def resolve(rr, _):
                r = gi * 16 + rr
                fc_here = plsc.load_gather(flags, [r + lanes * 0])

                @pl.when(fc_here[0] < 0.0)
                def _():
                    pltpu.sync_copy(
                        cl_h.at[img, qs // 8 + r // 8, pl.ds(1, 31), r % 8],
                        rowbuf)
                    accs = tuple(rowbuf[0, pl.ds(k * 16, 16)] for k in range(8))

                    def chunk(t, a):
                        return tuple(
                            jnp.maximum(a[k], rowbuf[t, pl.ds(k * 16, 16)])
                            for k in range(8))

                    accs = lax.fori_loop(1, 31, chunk, accs)
                    m = accs[0]
                    for k in range(1, 8):
                        m = jnp.maximum(m, accs[k])
                    v0v = plsc.load_gather(
                        buf2, [lanes * 0 + r // 8, lanes * 0 + r % 8, lanes * 0])
                    val = jnp.where(jnp.max(m) > v0v[0], 1.0, 0.0) + lanes * 0.0
                    plsc.store_scatter(flags, [r + lanes * 0], val,
                                       mask=lanes == 0)
                return 0

            lax.fori_loop(0, 16, resolve, 0)
        return 0

    lax.fori_loop(0, NG, resolve_group, 0)

    pltpu.sync_copy(flags, out_h.at[pl.ds(img * 1024 + qs, QPW)])


def _tc_body(ts, bb, lb, ch, blg, llg, cf, out):
    for b in range(B):
        h = ts[0, b]
        w = ts[1, b]

        def keep(ref, b=b):
            v0 = ref[b, 0, :]
            acc = ref[b, 1, :] > v0
            for c in range(2, 16):
                acc = jnp.logical_or(acc, ref[b, c, :] > v0)
            return jnp.where(acc, 1.0, 0.0)

        fb = keep(blg)
        fl = keep(llg)
        fc = cf[8 * b:8 * b + 8, :].reshape(1024)[:Q]
        for base, src, f in ((0, bb, fb), (4, lb, fl)):
            cx = src[b, 0, :]
            cy = src[b, 1, :]
            hw = src[b, 2, :] * 0.5
            hh = src[b, 3, :] * 0.5
            out[b, base + 0, :] = (cx - hw) * w * f
            out[b, base + 1, :] = (cy - hh) * h * f
            out[b, base + 2, :] = (cx + hw) * w * f
            out[b, base + 3, :] = (cy + hh) * h * f
        for c in range(16):
            s = h if c % 2 == 0 else w
            out[b, 8 + c, :] = ch[b, c, :] * s * fc


@jax.jit
def kernel(pred_block, pred_line, pred_char, pred_block_logits,
           pred_line_logits, pred_char_logits, target_sizes):
    mesh = plsc.VectorSubcoreMesh(core_axis_name="c", subcore_axis_name="s")
    sc_run = functools.partial(
        pl.kernel,
        mesh=mesh,
        compiler_params=pltpu.CompilerParams(
            needs_layout_passes=False, use_tc_tiling_on_sc=False),
        out_type=jax.ShapeDtypeStruct((B * 1024,), jnp.float32),
        scratch_types=[
            pltpu.VMEM((QPW // 8, 8, 129), jnp.float32),
            pltpu.VMEM((31, 128), jnp.float32),
            pltpu.VMEM((QPW,), jnp.float32),
            pltpu.SemaphoreType.DMA,
        ],
    )(_sc_body)
    cflags = sc_run(
        pred_char_logits.reshape(B, Q // 8, 8, C // 128, 128)
                        .transpose(0, 1, 3, 2, 4)).reshape(B * 8, 128)

    out = pl.pallas_call(
        _tc_body,
        out_shape=jax.ShapeDtypeStruct((B, 24, Q), jnp.float32),
    )(target_sizes.transpose(1, 0),
      pred_block.transpose(0, 2, 1), pred_line.transpose(0, 2, 1),
      pred_char.transpose(0, 2, 1),
      pred_block_logits.transpose(0, 2, 1),
      pred_line_logits.transpose(0, 2, 1),
      cflags)
    return out.transpose(0, 2, 1)


# speedup vs baseline: 1.9562x; 1.0832x over previous
import functools

import jax
import jax.numpy as jnp
from jax import lax
from jax.experimental import pallas as pl
from jax.experimental.pallas import tpu as pltpu
from jax.experimental.pallas import tpu_sc as plsc

B, Q, C = 8, 1000, 4096
QPW = 256
QSTEP = 248
NG = QPW // 16


NSLOT = 8


def _sc_body(cl_h, out_h, buf2, rowbufs, flags, unres, sem, sem2):
    wid = lax.axis_index("s") * 2 + lax.axis_index("c")
    img = wid // 4
    qs = (wid % 4) * QSTEP

    half = QPW // 16
    cp0 = pltpu.async_copy(
        cl_h.at[img, pl.ds(qs // 8, half), 0],
        buf2.at[pl.ds(0, half), :, pl.ds(0, 128)], sem)
    cp1 = pltpu.async_copy(
        cl_h.at[img, pl.ds(qs // 8 + half, half), 0],
        buf2.at[pl.ds(half, half), :, pl.ds(0, 128)], sem2)

    lanes = lax.iota(jnp.int32, 16)

    zero16 = lanes * 0

    def char_group(gi, _):
        rows = gi * 16 + lanes
        tq = rows // 8
        qi = rows % 8
        v0 = plsc.load_gather(buf2, [tq, qi, zero16])
        accs = [v0 != v0] * 4
        for c0 in range(0, 128, 4):
            for k in range(4):
                cvec = jnp.full((16,), c0 + k, jnp.int32)
                accs[k] = jnp.logical_or(
                    accs[k], plsc.load_gather(buf2, [tq, qi, cvec]) > v0)
        acc = jnp.logical_or(jnp.logical_or(accs[0], accs[1]),
                             jnp.logical_or(accs[2], accs[3]))
        flags[pl.ds(gi * 16, 16)] = jnp.where(acc, 1.0, -1.0)
        return 0

    cp0.wait()
    lax.fori_loop(0, NG // 2, char_group, 0)
    cp1.wait()
    lax.fori_loop(NG // 2, NG, char_group, 0)

    def straggler_dma(r, slot, s):
        return pltpu.async_copy(
            cl_h.at[img, qs // 8 + r // 8, pl.ds(1, 31), r % 8],
            rowbufs.at[slot], s)

    def issue_group(gi, cnt):
        fvec = flags[pl.ds(gi * 16, 16)]

        def issue_row(rr, cnt):
            r = gi * 16 + rr
            f = plsc.load_gather(flags, [r + lanes * 0])
            go = jnp.logical_and(f[0] < 0.0, cnt < NSLOT)

            @pl.when(go)
            def _():
                straggler_dma(r, cnt, sem)
                plsc.store_scatter(unres, [cnt + lanes * 0], r + lanes * 0,
                                   mask=lanes == 0)
            return cnt + go.astype(jnp.int32)

        return lax.cond(jnp.min(fvec) < 0.0,
                        lambda c: lax.fori_loop(0, 16, issue_row, c),
                        lambda c: c, cnt)

    cnt = lax.fori_loop(0, NG, issue_group, 0)

    def drain(i, _):
        pltpu.make_async_copy(
            cl_h.at[img, 0, pl.ds(1, 31), 0], rowbufs.at[0], sem).wait()
        return 0

    lax.fori_loop(0, cnt, drain, 0)

    def scan_rowbuf(ref, i, r):
        accs = tuple(ref[i, 0, pl.ds(k * 16, 16)] for k in range(8))

        def chunk(t, a):
            return tuple(jnp.maximum(a[k], ref[i, t, pl.ds(k * 16, 16)])
                         for k in range(8))

        accs = lax.fori_loop(1, 31, chunk, accs)
        m = accs[0]
        for k in range(1, 8):
            m = jnp.maximum(m, accs[k])
        v0v = plsc.load_gather(
            buf2, [lanes * 0 + r // 8, lanes * 0 + r % 8, lanes * 0])
        val = jnp.where(jnp.max(m) > v0v[0], 1.0, 0.0) + lanes * 0.0
        plsc.store_scatter(flags, [r + lanes * 0], val, mask=lanes == 0)

    def scan_slot(i, _):
        r = plsc.load_gather(unres, [i + lanes * 0])[0]
        scan_rowbuf(rowbufs, i, r)
        return 0

    lax.fori_loop(0, cnt, scan_slot, 0)

    @pl.when(cnt >= NSLOT)
    def _():
        def resolve(r, _):
            f = plsc.load_gather(flags, [r + lanes * 0])

            @pl.when(f[0] < 0.0)
            def _():
                straggler_dma(r, 0, sem).wait()
                scan_rowbuf(rowbufs, 0, r)
            return 0

        lax.fori_loop(0, QPW, resolve, 0)

    pltpu.sync_copy(flags, out_h.at[pl.ds(img * 1024 + qs, QPW)])


def _tc_body(ts, bb, lb, ch, blg, llg, cf, out):
    for b in range(B):
        h = ts[0, b]
        w = ts[1, b]

        def keep(ref, b=b):
            v0 = ref[b, 0, :]
            acc = ref[b, 1, :] > v0
            for c in range(2, 16):
                acc = jnp.logical_or(acc, ref[b, c, :] > v0)
            return jnp.where(acc, 1.0, 0.0)

        fb = keep(blg)
        fl = keep(llg)
        fc = cf[8 * b:8 * b + 8, :].reshape(1024)[:Q]
        for base, src, f in ((0, bb, fb), (4, lb, fl)):
            cx = src[b, 0, :]
            cy = src[b, 1, :]
            hw = src[b, 2, :] * 0.5
            hh = src[b, 3, :] * 0.5
            out[b, base + 0, :] = (cx - hw) * w * f
            out[b, base + 1, :] = (cy - hh) * h * f
            out[b, base + 2, :] = (cx + hw) * w * f
            out[b, base + 3, :] = (cy + hh) * h * f
        for c in range(16):
            s = h if c % 2 == 0 else w
            out[b, 8 + c, :] = ch[b, c, :] * s * fc


@jax.jit
def kernel(pred_block, pred_line, pred_char, pred_block_logits,
           pred_line_logits, pred_char_logits, target_sizes):
    mesh = plsc.VectorSubcoreMesh(core_axis_name="c", subcore_axis_name="s")
    sc_run = functools.partial(
        pl.kernel,
        mesh=mesh,
        compiler_params=pltpu.CompilerParams(
            needs_layout_passes=False, use_tc_tiling_on_sc=False),
        out_type=jax.ShapeDtypeStruct((B * 1024,), jnp.float32),
        scratch_types=[
            pltpu.VMEM((QPW // 8, 8, 129), jnp.float32),
            pltpu.VMEM((NSLOT, 31, 128), jnp.float32),
            pltpu.VMEM((QPW,), jnp.float32),
            pltpu.VMEM((16,), jnp.int32),
            pltpu.SemaphoreType.DMA,
            pltpu.SemaphoreType.DMA,
        ],
    )(_sc_body)
    cflags = sc_run(
        pred_char_logits.reshape(B, Q // 8, 8, C // 128, 128)
                        .transpose(0, 1, 3, 2, 4)).reshape(B * 8, 128)

    out = pl.pallas_call(
        _tc_body,
        out_shape=jax.ShapeDtypeStruct((B, 24, Q), jnp.float32),
    )(target_sizes.transpose(1, 0),
      pred_block.transpose(0, 2, 1), pred_line.transpose(0, 2, 1),
      pred_char.transpose(0, 2, 1),
      pred_block_logits.transpose(0, 2, 1),
      pred_line_logits.transpose(0, 2, 1),
      cflags)
    return out.transpose(0, 2, 1)
